# Initial kernel scaffold; baseline (speedup 1.0000x reference)
import jax
import jax.numpy as jnp
from jax import lax
from jax.experimental import pallas as pl
from jax.experimental.pallas import tpu as pltpu
from jax.experimental.pallas import tpu_sc as plsc

N_NODES = 10000
N_PAD = 10240
D = 128
N_EDGES = 320000
NC, NS = 2, 16
CHUNKS, CHUNK = 125, 80
ROWS_PER_TILE = N_PAD // NS
BLK = 1000


def _mesh():
    return plsc.VectorSubcoreMesh(
        core_axis_name="c", subcore_axis_name="s", num_cores=NC, num_subcores=NS
    )


def _deg_body(dst_hbm, z1_hbm, deg_out, dsti_v, ones_v, deg_sh):
    cid = lax.axis_index("c")
    sid = lax.axis_index("s")
    base = pl.multiple_of(sid * ROWS_PER_TILE, ROWS_PER_TILE)
    pltpu.sync_copy(
        z1_hbm.at[pl.ds(base, ROWS_PER_TILE)], deg_sh.at[pl.ds(base, ROWS_PER_TILE)]
    )
    pltpu.sync_copy(dst_hbm.at[cid, sid], dsti_v)

    @pl.loop(0, CHUNK // 16)
    def _(i):
        ones_v[pl.ds(i * 16, 16)] = jnp.full((16,), 1.0, jnp.float32)

    plsc.subcore_barrier()

    @pl.loop(0, CHUNKS)
    def _(j):
        pltpu.sync_copy(ones_v, deg_sh.at[dsti_v.at[j]], add=True)

    plsc.subcore_barrier()
    pltpu.sync_copy(
        deg_sh.at[pl.ds(base, ROWS_PER_TILE)],
        deg_out.at[cid, pl.ds(base, ROWS_PER_TILE)],
    )


def _sc_deg(dst, z1):
    fn = pl.kernel(
        _deg_body,
        out_type=jax.ShapeDtypeStruct((NC, N_PAD), jnp.float32),
        mesh=_mesh(),
        scratch_types=[
            pltpu.VMEM((CHUNKS, CHUNK), jnp.int32),
            pltpu.VMEM((CHUNK,), jnp.float32),
            pltpu.VMEM_SHARED((N_PAD,), jnp.float32),
        ],
    )
    return fn(dst, z1)


def _prop_body(
    h2_hbm, src_hbm, dst_hbm, z2_hbm, acc_out,
    srci_v, dsti_v, rows0, rows1, sem0, sem1, acc_sh,
):
    cid = lax.axis_index("c")
    sid = lax.axis_index("s")
    base = pl.multiple_of(sid * ROWS_PER_TILE, ROWS_PER_TILE)
    pltpu.sync_copy(
        z2_hbm.at[pl.ds(base, ROWS_PER_TILE), :],
        acc_sh.at[pl.ds(base, ROWS_PER_TILE), :],
    )
    pltpu.sync_copy(src_hbm.at[cid, sid], srci_v)
    pltpu.sync_copy(dst_hbm.at[cid, sid], dsti_v)
    plsc.subcore_barrier()

    pltpu.async_copy(h2_hbm.at[srci_v.at[0]], rows0, sem0)

    @pl.loop(0, (CHUNKS - 1) // 2)
    def _(t):
        j0 = 2 * t
        pltpu.async_copy(h2_hbm.at[srci_v.at[j0 + 1]], rows1, sem1)
        pltpu.make_async_copy(h2_hbm.at[srci_v.at[j0]], rows0, sem0).wait()
        pltpu.sync_copy(rows0, acc_sh.at[dsti_v.at[j0]], add=True)
        pltpu.async_copy(h2_hbm.at[srci_v.at[j0 + 2]], rows0, sem0)
        pltpu.make_async_copy(h2_hbm.at[srci_v.at[j0 + 1]], rows1, sem1).wait()
        pltpu.sync_copy(rows1, acc_sh.at[dsti_v.at[j0 + 1]], add=True)

    pltpu.make_async_copy(h2_hbm.at[srci_v.at[CHUNKS - 1]], rows0, sem0).wait()
    pltpu.sync_copy(rows0, acc_sh.at[dsti_v.at[CHUNKS - 1]], add=True)

    plsc.subcore_barrier()
    pltpu.sync_copy(
        acc_sh.at[pl.ds(base, ROWS_PER_TILE), :],
        acc_out.at[cid, pl.ds(base, ROWS_PER_TILE), :],
    )


def _sc_prop(h2, src, dst, z2):
    fn = pl.kernel(
        _prop_body,
        out_type=jax.ShapeDtypeStruct((NC, N_PAD, D), jnp.float32),
        mesh=_mesh(),
        scratch_types=[
            pltpu.VMEM((CHUNKS, CHUNK), jnp.int32),
            pltpu.VMEM((CHUNKS, CHUNK), jnp.int32),
            pltpu.VMEM((CHUNK, D), jnp.float32),
            pltpu.VMEM((CHUNK, D), jnp.float32),
            pltpu.SemaphoreType.DMA,
            pltpu.SemaphoreType.DMA,
            pltpu.VMEM_SHARED((N_PAD, D), jnp.float32),
        ],
    )
    return fn(h2, src, dst, z2)


def _xform_body(x_ref, w_ref, d0_ref, d1_ref, h2_ref, dis_ref):
    deg = d0_ref[...] + d1_ref[...] + 1.0
    dis = lax.rsqrt(deg)
    dis_ref[...] = dis
    h = jnp.dot(x_ref[...], w_ref[...], preferred_element_type=jnp.float32)
    h2_ref[...] = h * dis


def _tc_xform(x, W, d0, d1):
    return pl.pallas_call(
        _xform_body,
        grid=(N_NODES // BLK,),
        in_specs=[
            pl.BlockSpec((BLK, D), lambda i: (i, 0)),
            pl.BlockSpec((D, D), lambda i: (0, 0)),
            pl.BlockSpec((BLK, 1), lambda i: (i, 0)),
            pl.BlockSpec((BLK, 1), lambda i: (i, 0)),
        ],
        out_specs=[
            pl.BlockSpec((BLK, D), lambda i: (i, 0)),
            pl.BlockSpec((BLK, 1), lambda i: (i, 0)),
        ],
        out_shape=[
            jax.ShapeDtypeStruct((N_NODES, D), jnp.float32),
            jax.ShapeDtypeStruct((N_NODES, 1), jnp.float32),
        ],
    )(x, W, d0, d1)


def _final_body(a0_ref, a1_ref, h2_ref, dis_ref, b_ref, out_ref):
    s = a0_ref[...] + a1_ref[...] + h2_ref[...]
    out_ref[...] = jnp.maximum(s * dis_ref[...] + b_ref[...], 0.0)


def _tc_final(a0, a1, h2, dis, b2):
    return pl.pallas_call(
        _final_body,
        grid=(N_NODES // BLK,),
        in_specs=[
            pl.BlockSpec((BLK, D), lambda i: (i, 0)),
            pl.BlockSpec((BLK, D), lambda i: (i, 0)),
            pl.BlockSpec((BLK, D), lambda i: (i, 0)),
            pl.BlockSpec((BLK, 1), lambda i: (i, 0)),
            pl.BlockSpec((1, D), lambda i: (0, 0)),
        ],
        out_specs=pl.BlockSpec((BLK, D), lambda i: (i, 0)),
        out_shape=jax.ShapeDtypeStruct((N_NODES, D), jnp.float32),
    )(a0, a1, h2, dis, b2)


@jax.jit
def _impl(x, edge_index, W, b):
    ei = edge_index.astype(jnp.int32)
    src = ei[0].reshape(NC, NS, CHUNKS, CHUNK)
    dst = ei[1].reshape(NC, NS, CHUNKS, CHUNK)
    z1 = jnp.zeros((N_PAD,), jnp.float32)
    z2 = jnp.zeros((N_PAD, D), jnp.float32)

    deg_parts = _sc_deg(dst, z1)
    dp = deg_parts[:, :N_NODES, None]
    h2, dis = _tc_xform(x, W, dp[0], dp[1])
    acc = _sc_prop(h2, src, dst, z2)
    return _tc_final(acc[0, :N_NODES], acc[1, :N_NODES], h2, dis, b.reshape(1, D))


def kernel(x, edge_index, W, b):
    return _impl(x, edge_index, W, b)


# speedup vs baseline: 35.4579x; 35.4579x over previous
import jax
import jax.numpy as jnp
from jax import lax
from jax.experimental import pallas as pl
from jax.experimental.pallas import tpu as pltpu
from jax.experimental.pallas import tpu_sc as plsc

N_NODES = 10000
N_PAD = 10240
D = 128
N_EDGES = 320000
NC, NS = 2, 16
CHUNKS, CHUNK = 125, 80
GROUPS = 5
GCHUNKS = CHUNKS // GROUPS
ROWS_PER_TILE = N_PAD // NS
BLK = 1000


def _mesh():
    return plsc.VectorSubcoreMesh(
        core_axis_name="c", subcore_axis_name="s", num_cores=NC, num_subcores=NS
    )


def _deg_body(dst_hbm, z1_hbm, deg_out, dsti_v, ones_v, deg_sh):
    cid = lax.axis_index("c")
    sid = lax.axis_index("s")
    base = pl.multiple_of(sid * ROWS_PER_TILE, ROWS_PER_TILE)
    pltpu.sync_copy(
        z1_hbm.at[pl.ds(base, ROWS_PER_TILE)], deg_sh.at[pl.ds(base, ROWS_PER_TILE)]
    )
    pltpu.sync_copy(dst_hbm.at[cid, sid], dsti_v)

    @pl.loop(0, CHUNK // 16)
    def _(i):
        ones_v[pl.ds(i * 16, 16)] = jnp.full((16,), 1.0, jnp.float32)

    plsc.subcore_barrier()

    @pl.loop(0, CHUNKS)
    def _(j):
        pltpu.sync_copy(ones_v, deg_sh.at[dsti_v.at[j]], add=True)

    plsc.subcore_barrier()
    pltpu.sync_copy(
        deg_sh.at[pl.ds(base, ROWS_PER_TILE)],
        deg_out.at[cid, pl.ds(base, ROWS_PER_TILE)],
    )


def _sc_deg(dst, z1):
    fn = pl.kernel(
        _deg_body,
        out_type=jax.ShapeDtypeStruct((NC, N_PAD), jnp.float32),
        mesh=_mesh(),
        scratch_types=[
            pltpu.VMEM((CHUNKS, CHUNK), jnp.int32),
            pltpu.VMEM((CHUNK,), jnp.float32),
            pltpu.VMEM_SHARED((N_PAD,), jnp.float32),
        ],
    )
    return fn(dst, z1)


def _prop_body(
    h2_hbm, src_hbm, dst_hbm, z2_hbm, acc_out,
    srci_v, dsti_v, rows0, rows1, sem0, sem1, acc_sh,
):
    cid = lax.axis_index("c")
    sid = lax.axis_index("s")
    base = pl.multiple_of(sid * ROWS_PER_TILE, ROWS_PER_TILE)
    pltpu.sync_copy(
        z2_hbm.at[pl.ds(base, ROWS_PER_TILE), :],
        acc_sh.at[pl.ds(base, ROWS_PER_TILE), :],
    )
    plsc.subcore_barrier()

    @pl.loop(0, GROUPS)
    def _(g):
        pltpu.sync_copy(src_hbm.at[cid, sid, g], srci_v)
        pltpu.sync_copy(dst_hbm.at[cid, sid, g], dsti_v)
        pltpu.async_copy(h2_hbm.at[srci_v.at[0]], rows0, sem0)

        @pl.loop(0, (GCHUNKS - 1) // 2)
        def _(t):
            j0 = 2 * t
            pltpu.async_copy(h2_hbm.at[srci_v.at[j0 + 1]], rows1, sem1)
            pltpu.make_async_copy(h2_hbm.at[srci_v.at[j0]], rows0, sem0).wait()
            pltpu.sync_copy(rows0, acc_sh.at[dsti_v.at[j0]], add=True)
            pltpu.async_copy(h2_hbm.at[srci_v.at[j0 + 2]], rows0, sem0)
            pltpu.make_async_copy(h2_hbm.at[srci_v.at[j0 + 1]], rows1, sem1).wait()
            pltpu.sync_copy(rows1, acc_sh.at[dsti_v.at[j0 + 1]], add=True)

        pltpu.make_async_copy(h2_hbm.at[srci_v.at[GCHUNKS - 1]], rows0, sem0).wait()
        pltpu.sync_copy(rows0, acc_sh.at[dsti_v.at[GCHUNKS - 1]], add=True)

    plsc.subcore_barrier()
    pltpu.sync_copy(
        acc_sh.at[pl.ds(base, ROWS_PER_TILE), :],
        acc_out.at[cid, pl.ds(base, ROWS_PER_TILE), :],
    )


def _sc_prop(h2, src, dst, z2):
    fn = pl.kernel(
        _prop_body,
        out_type=jax.ShapeDtypeStruct((NC, N_PAD, D), jnp.float32),
        mesh=_mesh(),
        scratch_types=[
            pltpu.VMEM((GCHUNKS, CHUNK), jnp.int32),
            pltpu.VMEM((GCHUNKS, CHUNK), jnp.int32),
            pltpu.VMEM((CHUNK, D), jnp.float32),
            pltpu.VMEM((CHUNK, D), jnp.float32),
            pltpu.SemaphoreType.DMA,
            pltpu.SemaphoreType.DMA,
            pltpu.VMEM_SHARED((N_PAD, D), jnp.float32),
        ],
    )
    return fn(h2, src, dst, z2)


def _xform_body(x_ref, w_ref, d0_ref, d1_ref, h2_ref, dis_ref):
    deg = d0_ref[...] + d1_ref[...] + 1.0
    dis = lax.rsqrt(deg)
    dis_ref[...] = dis
    h = jnp.dot(x_ref[...], w_ref[...], preferred_element_type=jnp.float32)
    h2_ref[...] = h * dis


def _tc_xform(x, W, d0, d1):
    return pl.pallas_call(
        _xform_body,
        grid=(N_NODES // BLK,),
        in_specs=[
            pl.BlockSpec((BLK, D), lambda i: (i, 0)),
            pl.BlockSpec((D, D), lambda i: (0, 0)),
            pl.BlockSpec((BLK, 1), lambda i: (i, 0)),
            pl.BlockSpec((BLK, 1), lambda i: (i, 0)),
        ],
        out_specs=[
            pl.BlockSpec((BLK, D), lambda i: (i, 0)),
            pl.BlockSpec((BLK, 1), lambda i: (i, 0)),
        ],
        out_shape=[
            jax.ShapeDtypeStruct((N_NODES, D), jnp.float32),
            jax.ShapeDtypeStruct((N_NODES, 1), jnp.float32),
        ],
    )(x, W, d0, d1)


def _final_body(a0_ref, a1_ref, h2_ref, dis_ref, b_ref, out_ref):
    s = a0_ref[...] + a1_ref[...] + h2_ref[...]
    out_ref[...] = jnp.maximum(s * dis_ref[...] + b_ref[...], 0.0)


def _tc_final(a0, a1, h2, dis, b2):
    return pl.pallas_call(
        _final_body,
        grid=(N_NODES // BLK,),
        in_specs=[
            pl.BlockSpec((BLK, D), lambda i: (i, 0)),
            pl.BlockSpec((BLK, D), lambda i: (i, 0)),
            pl.BlockSpec((BLK, D), lambda i: (i, 0)),
            pl.BlockSpec((BLK, 1), lambda i: (i, 0)),
            pl.BlockSpec((1, D), lambda i: (0, 0)),
        ],
        out_specs=pl.BlockSpec((BLK, D), lambda i: (i, 0)),
        out_shape=jax.ShapeDtypeStruct((N_NODES, D), jnp.float32),
    )(a0, a1, h2, dis, b2)


@jax.jit
def _impl(x, edge_index, W, b):
    ei = edge_index.astype(jnp.int32)
    src = ei[0].reshape(NC, NS, CHUNKS, CHUNK)
    dst = ei[1].reshape(NC, NS, CHUNKS, CHUNK)
    z1 = jnp.zeros((N_PAD,), jnp.float32)
    z2 = jnp.zeros((N_PAD, D), jnp.float32)

    deg_parts = _sc_deg(dst, z1)
    dp = deg_parts[:, :N_NODES, None]
    h2, dis = _tc_xform(x, W, dp[0], dp[1])
    src5 = src.reshape(NC, NS, GROUPS, GCHUNKS, CHUNK)
    dst5 = dst.reshape(NC, NS, GROUPS, GCHUNKS, CHUNK)
    acc = _sc_prop(h2, src5, dst5, z2)
    return _tc_final(acc[0, :N_NODES], acc[1, :N_NODES], h2, dis, b.reshape(1, D))


def kernel(x, edge_index, W, b):
    return _impl(x, edge_index, W, b)
